# Initial kernel scaffold; baseline (speedup 1.0000x reference)
import jax
import jax.numpy as jnp
from jax.experimental import pallas as pl


def kernel(position, angle, mass, velocity, from_bodies_position, to_bodies_position, stiffness, from_bodies, to_bodies):
    raise NotImplementedError("write your pallas kernel here")


# speedup vs baseline: 42.3188x; 42.3188x over previous
import functools

import jax
import jax.numpy as jnp
from jax import lax
from jax.experimental import pallas as pl
from jax.experimental.pallas import tpu as pltpu
from jax.experimental.pallas import tpu_sc as plsc

DT = 0.01
M_BODIES = 1000000
MP = 1048576
LANES = 128
ROWS = MP // LANES
KP = 1048576
IDX_ROWS = KP // 128

NC = 2
NS = 16
NW = NC * NS
ROWS_PER_W = IDX_ROWS // NW
HIST_PER_TILE = MP // NS
ZCHUNK = 4096


def _hist_body(idx_hbm, out_hbm, idx_v, ones_v, zeros_v, hist_sh):
    cid = lax.axis_index("c")
    sid = lax.axis_index("s")
    wid = sid * NC + cid

    for i in range(LANES // 16):
        ones_v[pl.ds(i * 16, 16)] = jnp.ones((16,), jnp.float32)
    for i in range(ZCHUNK // 16):
        zeros_v[pl.ds(i * 16, 16)] = jnp.zeros((16,), jnp.float32)

    for r in range(HIST_PER_TILE // ZCHUNK):
        pltpu.sync_copy(
            zeros_v, hist_sh.at[pl.ds(sid * HIST_PER_TILE + r * ZCHUNK, ZCHUNK)]
        )
    plsc.subcore_barrier()

    pltpu.sync_copy(idx_hbm.at[pl.ds(wid * ROWS_PER_W, ROWS_PER_W)], idx_v)

    def srow(j, carry):
        pltpu.sync_copy(ones_v, hist_sh.at[idx_v.at[j]], add=True)
        return carry

    lax.fori_loop(0, ROWS_PER_W, srow, 0)
    plsc.subcore_barrier()

    @pl.when(sid == 0)
    def _():
        pltpu.sync_copy(hist_sh, out_hbm.at[cid])


@functools.lru_cache(maxsize=1)
def _hist_call():
    return functools.partial(
        pl.kernel,
        mesh=plsc.VectorSubcoreMesh(core_axis_name="c", subcore_axis_name="s"),
        out_type=jax.ShapeDtypeStruct((NC, MP), jnp.float32),
        scratch_types=[
            pltpu.VMEM((ROWS_PER_W, 128), jnp.int32),
            pltpu.VMEM((LANES,), jnp.float32),
            pltpu.VMEM((ZCHUNK,), jnp.float32),
            pltpu.VMEM_SHARED((MP,), jnp.float32),
        ],
    )(_hist_body)


def _dense_a(px, py, rx, ry, ang, mass, c0, c1, absx, absy, hit, sums):
    step = pl.program_id(0)
    a = ang[...] - jnp.float32(jnp.pi / 2)
    c = jnp.cos(a)
    s = jnp.sin(a)
    rxv = rx[...]
    ryv = ry[...]
    ax = c * rxv - s * ryv + px[...]
    ay = s * rxv + c * ryv + py[...]
    cnt = c0[0] + c1[0]
    absx[...] = ax
    absy[...] = ay
    hit[...] = (cnt > 0.0).astype(jnp.float32)
    w = cnt * mass[...]

    @pl.when(step == 0)
    def _():
        sums[...] = jnp.zeros_like(sums)

    z = jnp.zeros((5, LANES), jnp.float32)
    sums[...] += jnp.concatenate(
        [
            jnp.sum(w * ax, axis=0, keepdims=True),
            jnp.sum(w * ay, axis=0, keepdims=True),
            jnp.sum(w, axis=0, keepdims=True),
            z,
        ],
        axis=0,
    )


def _dense_b(sums, stiff, vx, vy, absx, absy, hit, nvx, nvy):
    s = sums[...]
    wsum = jnp.sum(s[2, :])
    tx = jnp.sum(s[0, :]) / wsum
    ty = jnp.sum(s[1, :]) / wsum
    k = hit[...] * (stiff[0, 0] * DT)
    nvx[...] = vx[...] + k * (tx - absx[...])
    nvy[...] = vy[...] + k * (ty - absy[...])


def kernel(position, angle, mass, velocity, from_bodies_position,
           to_bodies_position, stiffness, from_bodies, to_bodies):
    f32 = jnp.float32
    idx = jnp.concatenate([from_bodies, to_bodies])
    idx = jnp.pad(idx, (0, KP - idx.shape[0]), constant_values=M_BODIES)
    idx2d = idx.reshape(IDX_ROWS, 128)

    pad1 = (0, MP - M_BODIES)

    def view1(x):
        return jnp.pad(x, pad1).reshape(ROWS, LANES)

    rel = jnp.concatenate([from_bodies_position, to_bodies_position])
    px = view1(position[:, 0])
    py = view1(position[:, 1])
    rx = view1(rel[:, 0])
    ry = view1(rel[:, 1])
    vx = view1(velocity[:, 0])
    vy = view1(velocity[:, 1])
    angv = view1(angle)
    massv = view1(mass)

    cnt = _hist_call()(idx2d)
    cnt3 = cnt.reshape(NC, ROWS, LANES)

    BR = 512
    gsteps = ROWS // BR
    row_spec = pl.BlockSpec((BR, LANES), lambda i: (i, 0))
    sums_spec = pl.BlockSpec((8, LANES), lambda i: (0, 0))
    absx, absy, hit, sums = pl.pallas_call(
        _dense_a,
        grid=(gsteps,),
        in_specs=[row_spec] * 6 + [
            pl.BlockSpec((1, BR, LANES), lambda i: (0, i, 0)),
            pl.BlockSpec((1, BR, LANES), lambda i: (1, i, 0)),
        ],
        out_specs=[row_spec, row_spec, row_spec, sums_spec],
        out_shape=[
            jax.ShapeDtypeStruct((ROWS, LANES), f32),
            jax.ShapeDtypeStruct((ROWS, LANES), f32),
            jax.ShapeDtypeStruct((ROWS, LANES), f32),
            jax.ShapeDtypeStruct((8, LANES), f32),
        ],
    )(px, py, rx, ry, angv, massv, cnt3, cnt3)

    stiff2d = stiffness.reshape(1, 1).astype(f32)
    nvx, nvy = pl.pallas_call(
        _dense_b,
        grid=(gsteps,),
        in_specs=[
            sums_spec,
            pl.BlockSpec(memory_space=pltpu.SMEM),
            row_spec, row_spec, row_spec, row_spec, row_spec,
        ],
        out_specs=[row_spec, row_spec],
        out_shape=[
            jax.ShapeDtypeStruct((ROWS, LANES), f32),
            jax.ShapeDtypeStruct((ROWS, LANES), f32),
        ],
    )(sums, stiff2d, vx, vy, absx, absy, hit)

    new_velocity = jnp.stack(
        [nvx.reshape(MP)[:M_BODIES], nvy.reshape(MP)[:M_BODIES]], axis=1
    )
    return new_velocity
